# baseline (speedup 1.0000x reference)
import functools

import jax
import jax.numpy as jnp
from jax import lax
from jax.experimental import pallas as pl
from jax.experimental.pallas import tpu as pltpu
from jax.experimental.pallas import tpu_sc as plsc

_NB_ROWS = 512
_CH = 96

_DN_T = (((1,), (1,)), ((), ()))
_DN_N = (((1,), (0,)), ((), ()))


def _argmin_min(dist, kdim):
    m = jnp.min(dist, axis=1, keepdims=True)
    ids = lax.broadcasted_iota(jnp.int32, dist.shape, 1).astype(jnp.float32)
    idxf = jnp.min(jnp.where(dist == m, ids, float(kdim)), axis=1)
    return m, ids, idxf


def _fused_body(x_ref, cb0_ref, cb1_ref, idx0_ref, idx1_ref, loss_ref,
                e20_ref, e21_ref, *, kdim, grid, scale):
    i = pl.program_id(0)

    @pl.when(i == 0)
    def _():
        c0 = cb0_ref[...]
        e20_ref[...] = jnp.sum(c0 * c0, axis=1).reshape(1, kdim)
        c1 = cb1_ref[...]
        e21_ref[...] = jnp.sum(c1 * c1, axis=1).reshape(1, kdim)

    x = x_ref[...]
    x2 = jnp.sum(x * x, axis=1, keepdims=True)
    xe2 = lax.dot_general(x * -2.0, cb0_ref[...], _DN_T,
                          preferred_element_type=jnp.float32)
    dist0 = (x2 + e20_ref[...]) + xe2
    m0, ids, idx0f = _argmin_min(dist0, kdim)
    idx0_ref[0, 0, :] = idx0f.astype(jnp.int32)

    onehot = jnp.where(ids == idx0f[:, None], 1.0, 0.0)
    q0 = lax.dot_general(onehot, cb0_ref[...], _DN_N,
                         precision=lax.Precision.HIGHEST,
                         preferred_element_type=jnp.float32)

    r = x - q0
    r2 = jnp.sum(r * r, axis=1, keepdims=True)
    re2 = lax.dot_general(r * -2.0, cb1_ref[...], _DN_T,
                          preferred_element_type=jnp.float32)
    dist1 = (r2 + e21_ref[...]) + re2
    m1, _, idx1f = _argmin_min(dist1, kdim)
    idx1_ref[0, 0, :] = idx1f.astype(jnp.int32)

    s = jnp.sum(m0) + jnp.sum(m1)

    @pl.when(i == 0)
    def _():
        loss_ref[0, 0] = s

    @pl.when(i != 0)
    def _():
        loss_ref[0, 0] += s

    @pl.when(i == grid - 1)
    def _():
        loss_ref[0, 0] = 1.25 * loss_ref[0, 0] * scale


def _tc_fused(x, cb0, cb1):
    n, d = x.shape
    k = cb0.shape[0]
    nb = _NB_ROWS
    grid = n // nb
    row_spec = pl.BlockSpec((nb, d), lambda i: (i, 0))
    cb_spec = pl.BlockSpec((k, d), lambda i: (0, 0))
    idx_spec = pl.BlockSpec((1, 1, nb), lambda i: (i, 0, 0))
    idx_ty = jax.ShapeDtypeStruct((grid, 1, nb), jnp.int32)
    idx0, idx1, loss = pl.pallas_call(
        functools.partial(_fused_body, kdim=k, grid=grid,
                          scale=1.0 / float(n * d)),
        grid=(grid,),
        in_specs=[row_spec, cb_spec, cb_spec],
        out_specs=[
            idx_spec,
            idx_spec,
            pl.BlockSpec((1, 1), lambda i: (0, 0), memory_space=pltpu.SMEM),
        ],
        out_shape=[idx_ty, idx_ty, jax.ShapeDtypeStruct((1, 1), jnp.float32)],
        scratch_shapes=[pltpu.VMEM((1, k), jnp.float32),
                        pltpu.VMEM((1, k), jnp.float32)],
    )(x, cb0, cb1)
    return idx0.reshape(n), idx1.reshape(n), loss


def _sc_combine(cb0, cb1, idx0, idx1):
    info = plsc.get_sparse_core_info()
    ncores, nsub = info.num_cores, info.num_subcores
    nw = ncores * nsub
    n = idx0.shape[0]
    d = cb0.shape[1]
    rows_w = n // nw
    ch = _CH
    nch = rows_w // ch
    mesh = plsc.VectorSubcoreMesh(core_axis_name="c", subcore_axis_name="s")

    @functools.partial(
        pl.kernel,
        out_type=[
            jax.ShapeDtypeStruct((n, d), jnp.float32),
            jax.ShapeDtypeStruct((2, nw, nch, ch), jnp.int32),
        ],
        mesh=mesh,
        scratch_types=[
            pltpu.VMEM((nch, ch), jnp.int32),
            pltpu.VMEM((nch, ch), jnp.int32),
            pltpu.VMEM((2, ch, d), jnp.float32),
            pltpu.VMEM((2, ch, d), jnp.float32),
            [pltpu.SemaphoreType.DMA] * nch,
            [pltpu.SemaphoreType.DMA] * nch,
            [pltpu.SemaphoreType.DMA] * nch,
            pltpu.SemaphoreType.DMA,
        ],
    )
    def k(cb0_hbm, cb1_hbm, idx0_hbm, idx1_hbm, out_hbm, codes_hbm,
          idx0_v, idx1_v, rows0_v, rows1_v, g0sems, g1sems, wsems, csem):
        wid = lax.axis_index("s") * ncores + lax.axis_index("c")
        base = wid * rows_w
        pltpu.sync_copy(idx0_hbm.at[wid], idx0_v)
        pltpu.sync_copy(idx1_hbm.at[wid], idx1_v)
        cs = [
            pltpu.async_copy(idx0_v, codes_hbm.at[0, wid], csem),
            pltpu.async_copy(idx1_v, codes_hbm.at[1, wid], csem),
        ]

        def fire(c):
            g0 = pltpu.async_copy(cb0_hbm.at[idx0_v.at[c]], rows0_v.at[c % 2],
                                  g0sems[c])
            g1 = pltpu.async_copy(cb1_hbm.at[idx1_v.at[c]], rows1_v.at[c % 2],
                                  g1sems[c])
            return g0, g1

        inflight = [fire(0)]
        ws = []
        for c in range(nch):
            if c + 1 < nch:
                if c >= 1:
                    ws[c - 1].wait()
                inflight.append(fire(c + 1))
            g0, g1 = inflight[c]
            g0.wait()
            g1.wait()
            bb = c % 2

            def body(rr, carry):
                for j in range(d // 16):
                    sl = pl.ds(j * 16, 16)
                    plsc.addupdate(rows0_v.at[bb, rr, sl], rows1_v[bb, rr, sl])
                return carry

            lax.fori_loop(0, ch, body, 0)
            ws.append(pltpu.async_copy(
                rows0_v.at[bb], out_hbm.at[pl.ds(base + c * ch, ch)],
                wsems[c]))
        for w in ws[max(0, nch - 2):]:
            w.wait()
        for c0 in cs:
            c0.wait()

    return k(cb0, cb1, idx0.reshape(nw, nch, ch), idx1.reshape(nw, nch, ch))


def kernel(x, cb0, cb1):
    b, t, d = x.shape
    n = b * t
    xf = x.reshape(n, d)
    idx0, idx1, loss = _tc_fused(xf, cb0, cb1)
    qt, codes2 = _sc_combine(cb0, cb1, idx0, idx1)
    quantized = qt.reshape(b, t, d)
    codes = codes2.reshape(2, b, t)
    return quantized, codes, loss.reshape(())


# speedup vs baseline: 1.2629x; 1.2629x over previous
import functools

import jax
import jax.numpy as jnp
from jax import lax
from jax.experimental import pallas as pl
from jax.experimental.pallas import tpu as pltpu
from jax.experimental.pallas import tpu_sc as plsc

_NB_ROWS = 512
_CH = 96

_DN_T = (((1,), (1,)), ((), ()))


def _argmin_tail(dist, kdim, idx_ref):
    m = jnp.min(dist, axis=1, keepdims=True)
    ids = lax.broadcasted_iota(jnp.int32, dist.shape, 1).astype(jnp.float32)
    idx = jnp.min(jnp.where(dist == m, ids, float(kdim)), axis=1)
    idx_ref[0, 0, :] = idx.astype(jnp.int32)
    return jnp.sum(m)


def _stage0_body(e2_ref, x_ref, cb_ref, idx_ref, part_ref, *, kdim):
    x = x_ref[...]
    x2 = jnp.sum(x * x, axis=1, keepdims=True)
    xe2 = lax.dot_general(x * -2.0, cb_ref[...], _DN_T,
                          preferred_element_type=jnp.float32)
    dist = (x2 + e2_ref[...]) + xe2
    s = _argmin_tail(dist, kdim, idx_ref)
    i = pl.program_id(0)

    @pl.when(i == 0)
    def _():
        part_ref[0, 0] = s

    @pl.when(i != 0)
    def _():
        part_ref[0, 0] += s


def _stage1_body(e2_ref, x_ref, q0_ref, cb_ref, p0_ref, idx_ref, part_ref, *,
                 kdim, grid, scale):
    r = x_ref[...] - q0_ref[...]
    r2 = jnp.sum(r * r, axis=1, keepdims=True)
    xe2 = lax.dot_general(r * -2.0, cb_ref[...], _DN_T,
                          preferred_element_type=jnp.float32)
    dist = (r2 + e2_ref[...]) + xe2
    s = _argmin_tail(dist, kdim, idx_ref)
    i = pl.program_id(0)

    @pl.when(i == 0)
    def _():
        part_ref[0, 0] = s

    @pl.when(i != 0)
    def _():
        part_ref[0, 0] += s

    @pl.when(i == grid - 1)
    def _():
        part_ref[0, 0] = 1.25 * (part_ref[0, 0] + p0_ref[0, 0]) * scale


def _tc_stage(e2, x, q0, cb, p0):
    n, d = x.shape
    k = cb.shape[0]
    nb = _NB_ROWS
    grid = n // nb
    row_spec = pl.BlockSpec((nb, d), lambda i: (i, 0))
    smem_spec = pl.BlockSpec((1, 1), lambda i: (0, 0), memory_space=pltpu.SMEM)
    in_specs = [
        pl.BlockSpec((1, k), lambda i: (0, 0)),
        row_spec,
    ]
    args = [e2, x]
    if q0 is None:
        body = functools.partial(_stage0_body, kdim=k)
    else:
        body = functools.partial(_stage1_body, kdim=k, grid=grid,
                                 scale=1.0 / float(n * d))
        in_specs.append(row_spec)
        args.append(q0)
    in_specs.append(pl.BlockSpec((k, d), lambda i: (0, 0)))
    args.append(cb)
    if q0 is not None:
        in_specs.append(smem_spec)
        args.append(p0)
    idx, part = pl.pallas_call(
        body,
        grid=(grid,),
        in_specs=in_specs,
        out_specs=[
            pl.BlockSpec((1, 1, nb), lambda i: (i, 0, 0)),
            smem_spec,
        ],
        out_shape=[
            jax.ShapeDtypeStruct((grid, 1, nb), jnp.int32),
            jax.ShapeDtypeStruct((1, 1), jnp.float32),
        ],
    )(*args)
    return idx.reshape(n), part


def _sc_gather(cb, idx):
    info = plsc.get_sparse_core_info()
    ncores, nsub = info.num_cores, info.num_subcores
    nw = ncores * nsub
    n = idx.shape[0]
    d = cb.shape[1]
    rows_w = n // nw
    ch = _CH
    nch = rows_w // ch
    mesh = plsc.VectorSubcoreMesh(core_axis_name="c", subcore_axis_name="s")

    @functools.partial(
        pl.kernel,
        out_type=jax.ShapeDtypeStruct((n, d), jnp.float32),
        mesh=mesh,
        scratch_types=[
            pltpu.VMEM((nch, ch), jnp.int32),
            pltpu.VMEM((nch, ch, d), jnp.float32),
            [pltpu.SemaphoreType.DMA] * nch,
            [pltpu.SemaphoreType.DMA] * nch,
        ],
    )
    def k(cb_hbm, idx_hbm, out_hbm, idx_v, rows_v, gsems, wsems):
        wid = lax.axis_index("s") * ncores + lax.axis_index("c")
        base = wid * rows_w
        pltpu.sync_copy(idx_hbm.at[wid], idx_v)
        gs = [pltpu.async_copy(cb_hbm.at[idx_v.at[c]], rows_v.at[c], gsems[c])
              for c in range(nch)]
        ws = []
        for c in range(nch):
            gs[c].wait()
            ws.append(pltpu.async_copy(
                rows_v.at[c], out_hbm.at[pl.ds(base + c * ch, ch)], wsems[c]))
        for w in ws:
            w.wait()

    return k(cb, idx.reshape(nw, nch, ch))


def _sc_gather_add(cb, idx, prev, idx_prev):
    info = plsc.get_sparse_core_info()
    ncores, nsub = info.num_cores, info.num_subcores
    nw = ncores * nsub
    n = idx.shape[0]
    d = cb.shape[1]
    rows_w = n // nw
    ch = _CH
    nch = rows_w // ch
    mesh = plsc.VectorSubcoreMesh(core_axis_name="c", subcore_axis_name="s")

    @functools.partial(
        pl.kernel,
        out_type=[
            jax.ShapeDtypeStruct((n, d), jnp.float32),
            jax.ShapeDtypeStruct((2, nw, nch, ch), jnp.int32),
        ],
        mesh=mesh,
        scratch_types=[
            pltpu.VMEM((nch, ch), jnp.int32),
            pltpu.VMEM((nch, ch), jnp.int32),
            pltpu.VMEM((2, ch, d), jnp.float32),
            pltpu.VMEM((2, ch, d), jnp.float32),
            [pltpu.SemaphoreType.DMA] * nch,
            [pltpu.SemaphoreType.DMA] * nch,
            [pltpu.SemaphoreType.DMA] * nch,
            pltpu.SemaphoreType.DMA,
        ],
    )
    def k(cb_hbm, idx_hbm, prev_hbm, idxp_hbm, out_hbm, codes_hbm,
          idx_v, idxp_v, rows_v, acc_v, gsems, psems, wsems, csem):
        wid = lax.axis_index("s") * ncores + lax.axis_index("c")
        base = wid * rows_w
        pltpu.sync_copy(idx_hbm.at[wid], idx_v)
        pltpu.sync_copy(idxp_hbm.at[wid], idxp_v)
        cs = [
            pltpu.async_copy(idxp_v, codes_hbm.at[0, wid], csem),
            pltpu.async_copy(idx_v, codes_hbm.at[1, wid], csem),
        ]

        def fire(c):
            g = pltpu.async_copy(cb_hbm.at[idx_v.at[c]], rows_v.at[c % 2],
                                 gsems[c])
            p = pltpu.async_copy(prev_hbm.at[pl.ds(base + c * ch, ch)],
                                 acc_v.at[c % 2], psems[c])
            return g, p

        inflight = [fire(0)]
        ws = []
        for c in range(nch):
            if c + 1 < nch:
                if c >= 1:
                    ws[c - 1].wait()
                inflight.append(fire(c + 1))
            g, p = inflight[c]
            g.wait()
            p.wait()
            bb = c % 2

            def body(rr, carry):
                for j in range(d // 16):
                    sl = pl.ds(j * 16, 16)
                    plsc.addupdate(acc_v.at[bb, rr, sl], rows_v[bb, rr, sl])
                return carry

            lax.fori_loop(0, ch, body, 0)
            ws.append(pltpu.async_copy(
                acc_v.at[bb], out_hbm.at[pl.ds(base + c * ch, ch)], wsems[c]))
        for w in ws[max(0, nch - 2):]:
            w.wait()
        for c0 in cs:
            c0.wait()

    return k(cb, idx.reshape(nw, nch, ch), prev,
             idx_prev.reshape(nw, nch, ch))


def kernel(x, cb0, cb1):
    b, t, d = x.shape
    n = b * t
    xf = x.reshape(n, d)

    e2_0 = (cb0 ** 2).sum(axis=1)[None, :]
    idx0, part0 = _tc_stage(e2_0, xf, None, cb0, None)

    q0 = _sc_gather(cb0, idx0)

    e2_1 = (cb1 ** 2).sum(axis=1)[None, :]
    idx1, loss = _tc_stage(e2_1, xf, q0, cb1, part0)

    qt, codes2 = _sc_gather_add(cb1, idx1, q0, idx0)

    quantized = qt.reshape(b, t, d)
    codes = codes2.reshape(2, b, t)
    return quantized, codes, loss.reshape(())


# speedup vs baseline: 1.3042x; 1.0327x over previous
import functools

import jax
import jax.numpy as jnp
from jax import lax
from jax.experimental import pallas as pl
from jax.experimental.pallas import tpu as pltpu
from jax.experimental.pallas import tpu_sc as plsc

_NB_ROWS = 512
_CH = 96

_DN_T = (((1,), (1,)), ((), ()))


def _argmin_tail(dist, kdim, idx_ref):
    m = jnp.min(dist, axis=1, keepdims=True)
    ids = lax.broadcasted_iota(jnp.int32, dist.shape, 1).astype(jnp.float32)
    idx = jnp.min(jnp.where(dist == m, ids, float(kdim)), axis=1)
    idx_ref[0, 0, :] = idx.astype(jnp.int32)
    return jnp.sum(m)


def _stage0_body(e2_ref, x_ref, cb_ref, idx_ref, part_ref, *, kdim):
    x = x_ref[...]
    x2 = jnp.sum(x * x, axis=1, keepdims=True)
    xe2 = lax.dot_general(x * -2.0, cb_ref[...], _DN_T,
                          preferred_element_type=jnp.float32)
    dist = (x2 + e2_ref[...]) + xe2
    s = _argmin_tail(dist, kdim, idx_ref)
    i = pl.program_id(0)

    @pl.when(i == 0)
    def _():
        part_ref[0, 0] = s

    @pl.when(i != 0)
    def _():
        part_ref[0, 0] += s


def _stage1_body(e2_ref, x_ref, q0_ref, cb_ref, p0_ref, idx_ref, part_ref, *,
                 kdim, grid, scale):
    r = x_ref[...] - q0_ref[...]
    r2 = jnp.sum(r * r, axis=1, keepdims=True)
    xe2 = lax.dot_general(r * -2.0, cb_ref[...], _DN_T,
                          preferred_element_type=jnp.float32)
    dist = (r2 + e2_ref[...]) + xe2
    s = _argmin_tail(dist, kdim, idx_ref)
    i = pl.program_id(0)

    @pl.when(i == 0)
    def _():
        part_ref[0, 0] = s

    @pl.when(i != 0)
    def _():
        part_ref[0, 0] += s

    @pl.when(i == grid - 1)
    def _():
        part_ref[0, 0] = 1.25 * (part_ref[0, 0] + p0_ref[0, 0]) * scale


def _tc_stage(e2, x, q0, cb, p0):
    n, d = x.shape
    k = cb.shape[0]
    nb = _NB_ROWS
    grid = n // nb
    row_spec = pl.BlockSpec((nb, d), lambda i: (i, 0))
    smem_spec = pl.BlockSpec((1, 1), lambda i: (0, 0), memory_space=pltpu.SMEM)
    in_specs = [
        pl.BlockSpec((1, k), lambda i: (0, 0)),
        row_spec,
    ]
    args = [e2, x]
    if q0 is None:
        body = functools.partial(_stage0_body, kdim=k)
    else:
        body = functools.partial(_stage1_body, kdim=k, grid=grid,
                                 scale=1.0 / float(n * d))
        in_specs.append(row_spec)
        args.append(q0)
    in_specs.append(pl.BlockSpec((k, d), lambda i: (0, 0)))
    args.append(cb)
    if q0 is not None:
        in_specs.append(smem_spec)
        args.append(p0)
    idx, part = pl.pallas_call(
        body,
        grid=(grid,),
        in_specs=in_specs,
        out_specs=[
            pl.BlockSpec((1, 1, nb), lambda i: (i, 0, 0)),
            smem_spec,
        ],
        out_shape=[
            jax.ShapeDtypeStruct((grid, 1, nb), jnp.int32),
            jax.ShapeDtypeStruct((1, 1), jnp.float32),
        ],
    )(*args)
    return idx.reshape(n), part


def _sc_gather(cb, idx):
    info = plsc.get_sparse_core_info()
    ncores, nsub = info.num_cores, info.num_subcores
    nw = ncores * nsub
    n = idx.shape[0]
    d = cb.shape[1]
    rows_w = n // nw
    ch = _CH
    nch = rows_w // ch
    mesh = plsc.VectorSubcoreMesh(core_axis_name="c", subcore_axis_name="s")

    @functools.partial(
        pl.kernel,
        out_type=jax.ShapeDtypeStruct((n, d), jnp.float32),
        mesh=mesh,
        scratch_types=[
            pltpu.VMEM((ch,), jnp.int32),
            pltpu.VMEM((ch, d), jnp.float32),
            pltpu.SemaphoreType.DMA,
        ],
    )
    def k(cb_hbm, idx_hbm, out_hbm, idx_v, rows_v, sem):
        wid = lax.axis_index("s") * ncores + lax.axis_index("c")
        base = wid * rows_w
        for c in range(nch):
            off = base + c * ch
            pltpu.sync_copy(idx_hbm.at[pl.ds(off, ch)], idx_v)
            pltpu.async_copy(cb_hbm.at[idx_v], rows_v, sem).wait()
            pltpu.sync_copy(rows_v, out_hbm.at[pl.ds(off, ch)])

    return k(cb, idx)


def _sc_gather_add(cb, idx, prev):
    info = plsc.get_sparse_core_info()
    ncores, nsub = info.num_cores, info.num_subcores
    nw = ncores * nsub
    n = idx.shape[0]
    d = cb.shape[1]
    rows_w = n // nw
    ch = _CH
    nch = rows_w // ch
    mesh = plsc.VectorSubcoreMesh(core_axis_name="c", subcore_axis_name="s")

    @functools.partial(
        pl.kernel,
        out_type=jax.ShapeDtypeStruct((n, d), jnp.float32),
        mesh=mesh,
        scratch_types=[
            pltpu.VMEM((ch,), jnp.int32),
            pltpu.VMEM((ch, d), jnp.float32),
            pltpu.VMEM((ch, d), jnp.float32),
            pltpu.SemaphoreType.DMA,
        ],
    )
    def k(cb_hbm, idx_hbm, prev_hbm, out_hbm, idx_v, rows_v, acc_v, sem):
        wid = lax.axis_index("s") * ncores + lax.axis_index("c")
        base = wid * rows_w
        for c in range(nch):
            off = base + c * ch
            pltpu.sync_copy(idx_hbm.at[pl.ds(off, ch)], idx_v)
            cp = pltpu.async_copy(cb_hbm.at[idx_v], rows_v, sem)
            pltpu.sync_copy(prev_hbm.at[pl.ds(off, ch)], acc_v)
            cp.wait()

            def body(rr, carry):
                for j in range(d // 16):
                    sl = pl.ds(j * 16, 16)
                    plsc.addupdate(acc_v.at[rr, sl], rows_v[rr, sl])
                return carry

            lax.fori_loop(0, ch, body, 0)
            pltpu.sync_copy(acc_v, out_hbm.at[pl.ds(off, ch)])

    return k(cb, idx, prev)


def kernel(x, cb0, cb1):
    b, t, d = x.shape
    n = b * t
    xf = x.reshape(n, d)

    e2_0 = (cb0 ** 2).sum(axis=1)[None, :]
    idx0, part0 = _tc_stage(e2_0, xf, None, cb0, None)

    q0 = _sc_gather(cb0, idx0)

    e2_1 = (cb1 ** 2).sum(axis=1)[None, :]
    idx1, loss = _tc_stage(e2_1, xf, q0, cb1, part0)

    qt = _sc_gather_add(cb1, idx1, q0)

    quantized = qt.reshape(b, t, d)
    codes = jnp.stack([idx0.reshape(b, t), idx1.reshape(b, t)], axis=0)
    return quantized, codes, loss.reshape(())


# speedup vs baseline: 1.3508x; 1.0357x over previous
import functools

import jax
import jax.numpy as jnp
from jax import lax
from jax.experimental import pallas as pl
from jax.experimental.pallas import tpu as pltpu
from jax.experimental.pallas import tpu_sc as plsc

_NB_ROWS = 1024
_CH = 96

_DN_T = (((1,), (1,)), ((), ()))


def _argmin_tail(dist, kdim, idx_ref):
    m = jnp.min(dist, axis=1, keepdims=True)
    ids = lax.broadcasted_iota(jnp.int32, dist.shape, 1).astype(jnp.float32)
    idx = jnp.min(jnp.where(dist == m, ids, float(kdim)), axis=1)
    idx_ref[0, 0, :] = idx.astype(jnp.int32)
    return jnp.sum(m)


def _stage0_body(e2_ref, x_ref, cb_ref, idx_ref, part_ref, *, kdim):
    x = x_ref[...]
    x2 = jnp.sum(x * x, axis=1, keepdims=True)
    xe2 = lax.dot_general(x * -2.0, cb_ref[...], _DN_T,
                          preferred_element_type=jnp.float32)
    dist = (x2 + e2_ref[...]) + xe2
    s = _argmin_tail(dist, kdim, idx_ref)
    i = pl.program_id(0)

    @pl.when(i == 0)
    def _():
        part_ref[0, 0] = s

    @pl.when(i != 0)
    def _():
        part_ref[0, 0] += s


def _stage1_body(e2_ref, x_ref, q0_ref, cb_ref, p0_ref, idx_ref, part_ref, *,
                 kdim, grid, scale):
    r = x_ref[...] - q0_ref[...]
    r2 = jnp.sum(r * r, axis=1, keepdims=True)
    xe2 = lax.dot_general(r * -2.0, cb_ref[...], _DN_T,
                          preferred_element_type=jnp.float32)
    dist = (r2 + e2_ref[...]) + xe2
    s = _argmin_tail(dist, kdim, idx_ref)
    i = pl.program_id(0)

    @pl.when(i == 0)
    def _():
        part_ref[0, 0] = s

    @pl.when(i != 0)
    def _():
        part_ref[0, 0] += s

    @pl.when(i == grid - 1)
    def _():
        part_ref[0, 0] = 1.25 * (part_ref[0, 0] + p0_ref[0, 0]) * scale


def _tc_stage(e2, x, q0, cb, p0):
    n, d = x.shape
    k = cb.shape[0]
    nb = _NB_ROWS
    grid = n // nb
    row_spec = pl.BlockSpec((nb, d), lambda i: (i, 0))
    smem_spec = pl.BlockSpec((1, 1), lambda i: (0, 0), memory_space=pltpu.SMEM)
    in_specs = [
        pl.BlockSpec((1, k), lambda i: (0, 0)),
        row_spec,
    ]
    args = [e2, x]
    if q0 is None:
        body = functools.partial(_stage0_body, kdim=k)
    else:
        body = functools.partial(_stage1_body, kdim=k, grid=grid,
                                 scale=1.0 / float(n * d))
        in_specs.append(row_spec)
        args.append(q0)
    in_specs.append(pl.BlockSpec((k, d), lambda i: (0, 0)))
    args.append(cb)
    if q0 is not None:
        in_specs.append(smem_spec)
        args.append(p0)
    idx, part = pl.pallas_call(
        body,
        grid=(grid,),
        in_specs=in_specs,
        out_specs=[
            pl.BlockSpec((1, 1, nb), lambda i: (i, 0, 0)),
            smem_spec,
        ],
        out_shape=[
            jax.ShapeDtypeStruct((grid, 1, nb), jnp.int32),
            jax.ShapeDtypeStruct((1, 1), jnp.float32),
        ],
    )(*args)
    return idx.reshape(n), part


def _sc_gather(cb, idx):
    info = plsc.get_sparse_core_info()
    ncores, nsub = info.num_cores, info.num_subcores
    nw = ncores * nsub
    n = idx.shape[0]
    d = cb.shape[1]
    rows_w = n // nw
    ch = _CH
    nch = rows_w // ch
    mesh = plsc.VectorSubcoreMesh(core_axis_name="c", subcore_axis_name="s")

    @functools.partial(
        pl.kernel,
        out_type=jax.ShapeDtypeStruct((n, d), jnp.float32),
        mesh=mesh,
        scratch_types=[
            pltpu.VMEM((ch,), jnp.int32),
            pltpu.VMEM((ch, d), jnp.float32),
            pltpu.SemaphoreType.DMA,
        ],
    )
    def k(cb_hbm, idx_hbm, out_hbm, idx_v, rows_v, sem):
        wid = lax.axis_index("s") * ncores + lax.axis_index("c")
        base = wid * rows_w
        for c in range(nch):
            off = base + c * ch
            pltpu.sync_copy(idx_hbm.at[pl.ds(off, ch)], idx_v)
            pltpu.async_copy(cb_hbm.at[idx_v], rows_v, sem).wait()
            pltpu.sync_copy(rows_v, out_hbm.at[pl.ds(off, ch)])

    return k(cb, idx)


def _sc_gather_add(cb, idx, prev):
    info = plsc.get_sparse_core_info()
    ncores, nsub = info.num_cores, info.num_subcores
    nw = ncores * nsub
    n = idx.shape[0]
    d = cb.shape[1]
    rows_w = n // nw
    ch = _CH
    nch = rows_w // ch
    mesh = plsc.VectorSubcoreMesh(core_axis_name="c", subcore_axis_name="s")

    @functools.partial(
        pl.kernel,
        out_type=jax.ShapeDtypeStruct((n, d), jnp.float32),
        mesh=mesh,
        scratch_types=[
            pltpu.VMEM((ch,), jnp.int32),
            pltpu.VMEM((ch, d), jnp.float32),
            pltpu.VMEM((ch, d), jnp.float32),
            pltpu.SemaphoreType.DMA,
        ],
    )
    def k(cb_hbm, idx_hbm, prev_hbm, out_hbm, idx_v, rows_v, acc_v, sem):
        wid = lax.axis_index("s") * ncores + lax.axis_index("c")
        base = wid * rows_w
        for c in range(nch):
            off = base + c * ch
            pltpu.sync_copy(idx_hbm.at[pl.ds(off, ch)], idx_v)
            cp = pltpu.async_copy(cb_hbm.at[idx_v], rows_v, sem)
            pltpu.sync_copy(prev_hbm.at[pl.ds(off, ch)], acc_v)
            cp.wait()

            def body(rr, carry):
                for j in range(d // 16):
                    sl = pl.ds(j * 16, 16)
                    plsc.addupdate(acc_v.at[rr, sl], rows_v[rr, sl])
                return carry

            lax.fori_loop(0, ch, body, 0)
            pltpu.sync_copy(acc_v, out_hbm.at[pl.ds(off, ch)])

    return k(cb, idx, prev)


def kernel(x, cb0, cb1):
    b, t, d = x.shape
    n = b * t
    xf = x.reshape(n, d)

    e2_0 = (cb0 ** 2).sum(axis=1)[None, :]
    idx0, part0 = _tc_stage(e2_0, xf, None, cb0, None)

    q0 = _sc_gather(cb0, idx0)

    e2_1 = (cb1 ** 2).sum(axis=1)[None, :]
    idx1, loss = _tc_stage(e2_1, xf, q0, cb1, part0)

    qt = _sc_gather_add(cb1, idx1, q0)

    quantized = qt.reshape(b, t, d)
    codes = jnp.stack([idx0.reshape(b, t), idx1.reshape(b, t)], axis=0)
    return quantized, codes, loss.reshape(())


# speedup vs baseline: 1.3638x; 1.0096x over previous
import functools

import jax
import jax.numpy as jnp
from jax import lax
from jax.experimental import pallas as pl
from jax.experimental.pallas import tpu as pltpu
from jax.experimental.pallas import tpu_sc as plsc

_NB_ROWS = 2304
_CH = 96

_DN_T = (((1,), (1,)), ((), ()))


def _argmin_tail(dist, kdim, idx_ref):
    m = jnp.min(dist, axis=1, keepdims=True)
    ids = lax.broadcasted_iota(jnp.int32, dist.shape, 1).astype(jnp.float32)
    idx = jnp.min(jnp.where(dist == m, ids, float(kdim)), axis=1)
    idx_ref[0, 0, :] = idx.astype(jnp.int32)
    return jnp.sum(m)


def _stage0_body(e2_ref, x_ref, cb_ref, idx_ref, part_ref, *, kdim):
    x = x_ref[...]
    x2 = jnp.sum(x * x, axis=1, keepdims=True)
    xe2 = lax.dot_general(x * -2.0, cb_ref[...], _DN_T,
                          preferred_element_type=jnp.float32)
    dist = (x2 + e2_ref[...]) + xe2
    s = _argmin_tail(dist, kdim, idx_ref)
    i = pl.program_id(0)

    @pl.when(i == 0)
    def _():
        part_ref[0, 0] = s

    @pl.when(i != 0)
    def _():
        part_ref[0, 0] += s


def _stage1_body(e2_ref, x_ref, q0_ref, cb_ref, p0_ref, idx_ref, part_ref, *,
                 kdim, grid, scale):
    r = x_ref[...] - q0_ref[...]
    r2 = jnp.sum(r * r, axis=1, keepdims=True)
    xe2 = lax.dot_general(r * -2.0, cb_ref[...], _DN_T,
                          preferred_element_type=jnp.float32)
    dist = (r2 + e2_ref[...]) + xe2
    s = _argmin_tail(dist, kdim, idx_ref)
    i = pl.program_id(0)

    @pl.when(i == 0)
    def _():
        part_ref[0, 0] = s

    @pl.when(i != 0)
    def _():
        part_ref[0, 0] += s

    @pl.when(i == grid - 1)
    def _():
        part_ref[0, 0] = 1.25 * (part_ref[0, 0] + p0_ref[0, 0]) * scale


def _tc_stage(e2, x, q0, cb, p0):
    n, d = x.shape
    k = cb.shape[0]
    nb = _NB_ROWS
    grid = n // nb
    row_spec = pl.BlockSpec((nb, d), lambda i: (i, 0))
    smem_spec = pl.BlockSpec((1, 1), lambda i: (0, 0), memory_space=pltpu.SMEM)
    in_specs = [
        pl.BlockSpec((1, k), lambda i: (0, 0)),
        row_spec,
    ]
    args = [e2, x]
    if q0 is None:
        body = functools.partial(_stage0_body, kdim=k)
    else:
        body = functools.partial(_stage1_body, kdim=k, grid=grid,
                                 scale=1.0 / float(n * d))
        in_specs.append(row_spec)
        args.append(q0)
    in_specs.append(pl.BlockSpec((k, d), lambda i: (0, 0)))
    args.append(cb)
    if q0 is not None:
        in_specs.append(smem_spec)
        args.append(p0)
    idx, part = pl.pallas_call(
        body,
        grid=(grid,),
        in_specs=in_specs,
        out_specs=[
            pl.BlockSpec((1, 1, nb), lambda i: (i, 0, 0)),
            smem_spec,
        ],
        out_shape=[
            jax.ShapeDtypeStruct((grid, 1, nb), jnp.int32),
            jax.ShapeDtypeStruct((1, 1), jnp.float32),
        ],
    )(*args)
    return idx.reshape(n), part


def _sc_gather(cb, idx):
    info = plsc.get_sparse_core_info()
    ncores, nsub = info.num_cores, info.num_subcores
    nw = ncores * nsub
    n = idx.shape[0]
    d = cb.shape[1]
    rows_w = n // nw
    ch = _CH
    nch = rows_w // ch
    mesh = plsc.VectorSubcoreMesh(core_axis_name="c", subcore_axis_name="s")

    @functools.partial(
        pl.kernel,
        out_type=jax.ShapeDtypeStruct((n, d), jnp.float32),
        mesh=mesh,
        scratch_types=[
            pltpu.VMEM((ch,), jnp.int32),
            pltpu.VMEM((ch, d), jnp.float32),
            pltpu.SemaphoreType.DMA,
        ],
    )
    def k(cb_hbm, idx_hbm, out_hbm, idx_v, rows_v, sem):
        wid = lax.axis_index("s") * ncores + lax.axis_index("c")
        base = wid * rows_w
        for c in range(nch):
            off = base + c * ch
            pltpu.sync_copy(idx_hbm.at[pl.ds(off, ch)], idx_v)
            pltpu.async_copy(cb_hbm.at[idx_v], rows_v, sem).wait()
            pltpu.sync_copy(rows_v, out_hbm.at[pl.ds(off, ch)])

    return k(cb, idx)


def _sc_gather_add(cb, idx, prev):
    info = plsc.get_sparse_core_info()
    ncores, nsub = info.num_cores, info.num_subcores
    nw = ncores * nsub
    n = idx.shape[0]
    d = cb.shape[1]
    rows_w = n // nw
    ch = _CH
    nch = rows_w // ch
    mesh = plsc.VectorSubcoreMesh(core_axis_name="c", subcore_axis_name="s")

    @functools.partial(
        pl.kernel,
        out_type=jax.ShapeDtypeStruct((n, d), jnp.float32),
        mesh=mesh,
        scratch_types=[
            pltpu.VMEM((ch,), jnp.int32),
            pltpu.VMEM((ch, d), jnp.float32),
            pltpu.VMEM((ch, d), jnp.float32),
            pltpu.SemaphoreType.DMA,
        ],
    )
    def k(cb_hbm, idx_hbm, prev_hbm, out_hbm, idx_v, rows_v, acc_v, sem):
        wid = lax.axis_index("s") * ncores + lax.axis_index("c")
        base = wid * rows_w
        for c in range(nch):
            off = base + c * ch
            pltpu.sync_copy(idx_hbm.at[pl.ds(off, ch)], idx_v)
            cp = pltpu.async_copy(cb_hbm.at[idx_v], rows_v, sem)
            pltpu.sync_copy(prev_hbm.at[pl.ds(off, ch)], acc_v)
            cp.wait()

            def body(rr, carry):
                for j in range(d // 16):
                    sl = pl.ds(j * 16, 16)
                    plsc.addupdate(acc_v.at[rr, sl], rows_v[rr, sl])
                return carry

            lax.fori_loop(0, ch, body, 0)
            pltpu.sync_copy(acc_v, out_hbm.at[pl.ds(off, ch)])

    return k(cb, idx, prev)


def kernel(x, cb0, cb1):
    b, t, d = x.shape
    n = b * t
    xf = x.reshape(n, d)

    e2_0 = (cb0 ** 2).sum(axis=1)[None, :]
    idx0, part0 = _tc_stage(e2_0, xf, None, cb0, None)

    q0 = _sc_gather(cb0, idx0)

    e2_1 = (cb1 ** 2).sum(axis=1)[None, :]
    idx1, loss = _tc_stage(e2_1, xf, q0, cb1, part0)

    qt = _sc_gather_add(cb1, idx1, q0)

    quantized = qt.reshape(b, t, d)
    codes = jnp.stack([idx0.reshape(b, t), idx1.reshape(b, t)], axis=0)
    return quantized, codes, loss.reshape(())
